# Initial kernel scaffold; baseline (speedup 1.0000x reference)
import jax
import jax.numpy as jnp
from jax.experimental import pallas as pl


def kernel(nodes, features, prop_nodes, prop_features, max_prop_len, emb_table, W_gat, a_gat, W_sem, b_sem, q_sem, W_out, b_out, v, weight):
    raise NotImplementedError("write your pallas kernel here")


# speedup vs baseline: 21.6034x; 21.6034x over previous
import functools

import jax
import jax.numpy as jnp
from jax import lax
from jax.experimental import pallas as pl
from jax.experimental.pallas import tpu as pltpu
from jax.experimental.pallas import tpu_sc as plsc

_VOCAB = 32
_D = 16
_NT = 3
_NR = 8
_NW = 32
_LANES = 16


def _tables_body(emb_ref, wgat_ref, ar_ref, we_ref, aux_ref):
    emb = emb_ref[...]
    WE = jnp.dot(emb, wgat_ref[...], preferred_element_type=jnp.float32)
    we_ref[...] = WE
    f12 = lax.dot_general(ar_ref[...], WE, (((1,), (1,)), ((), ())),
                          preferred_element_type=jnp.float32)
    sel0 = (lax.broadcasted_iota(jnp.int32, (1, _D), 1) == 0).astype(jnp.float32)
    col0 = lax.dot_general(sel0, emb, (((1,), (1,)), ((), ())),
                           preferred_element_type=jnp.float32)
    aux_ref[...] = jnp.concatenate(
        [f12, col0, jnp.zeros((5, _VOCAB), jnp.float32)], axis=0)


def _make_sc_main(B):
    chunk = B // _NW
    ngrp = chunk // _LANES
    mesh = plsc.VectorSubcoreMesh(core_axis_name="c", subcore_axis_name="s")

    @functools.partial(
        pl.kernel,
        mesh=mesh,
        compiler_params=pltpu.CompilerParams(use_tc_tiling_on_sc=False,
                                             needs_layout_passes=False),
        out_type=[jax.ShapeDtypeStruct((2, _NT, _D, B), jnp.float32),
                  jax.ShapeDtypeStruct((2, _NT, _D, B), jnp.float32)],
        scratch_types=[
            pltpu.VMEM((2 * _NT * _NR, chunk), jnp.int32),
            pltpu.VMEM((2 * _NT * 3, chunk), jnp.int32),
            pltpu.VMEM((_VOCAB, _D), jnp.float32),
            pltpu.VMEM((8, _VOCAB), jnp.float32),
            pltpu.VMEM((2, _NT, _D, chunk), jnp.float32),
            pltpu.VMEM((2, _NT, _D, chunk), jnp.float32),
        ],
    )
    def sc_main(ids_hbm, cids_hbm, we_hbm, aux_hbm, z0_hbm, zm_hbm,
                ids_v, cids_v, we_v, aux_v, z0_v, zm_v):
        wid = lax.axis_index("s") * 2 + lax.axis_index("c")
        base = wid * chunk
        pltpu.sync_copy(ids_hbm.at[:, pl.ds(base, chunk)], ids_v)
        pltpu.sync_copy(cids_hbm.at[:, pl.ds(base, chunk)], cids_v)
        pltpu.sync_copy(we_hbm, we_v)
        pltpu.sync_copy(aux_hbm, aux_v)

        lane = lax.iota(jnp.int32, _LANES)
        zero16 = jnp.zeros((_LANES,), jnp.float32)

        def body(k, carry):
            g = k // (2 * _NT)
            it = k % (2 * _NT)
            i = it // _NT
            t = it % _NT
            col = g * _LANES + lane

            ids8 = [plsc.load_gather(ids_v, [jnp.full((_LANES,), it * _NR + r,
                                                      jnp.int32), col])
                    for r in range(_NR)]
            row0 = jnp.full((_LANES,), 0, jnp.int32)
            row1 = jnp.full((_LANES,), 1, jnp.int32)
            row2 = jnp.full((_LANES,), 2, jnp.int32)
            f1_0 = plsc.load_gather(aux_v, [row0, ids8[0]])
            f2 = [plsc.load_gather(aux_v, [row1, ids8[r]]) for r in range(_NR)]
            e = [jnp.where(x >= 0.0, x, 0.2 * x) for x in
                 [f1_0 + f2r for f2r in f2]]
            m = e[0]
            for r in range(1, _NR):
                m = jnp.maximum(m, e[r])
            ex = [jnp.exp(er - m) for er in e]
            s = ex[0]
            for r in range(1, _NR):
                s = s + ex[r]
            inv = 1.0 / s
            attn = [exr * inv for exr in ex]

            cacc = plsc.load_gather(aux_v, [row2, ids8[1]]) != 0.0
            for p in range(3):
                cid = plsc.load_gather(
                    cids_v, [jnp.full((_LANES,), it * 3 + p, jnp.int32), col])
                cacc = jnp.logical_or(
                    cacc, plsc.load_gather(aux_v, [row2, cid]) != 0.0)

            isp = jnp.full((_LANES,), i, jnp.int32)
            tsp = jnp.full((_LANES,), t, jnp.int32)
            for d in range(_D):
                dsp = jnp.full((_LANES,), d, jnp.int32)
                g0 = plsc.load_gather(we_v, [ids8[0], dsp])
                acc0 = attn[0] * g0
                accm = g0
                for r in range(1, _NR):
                    gr = plsc.load_gather(we_v, [ids8[r], dsp])
                    acc0 = acc0 + attn[r] * gr
                    accm = accm + gr
                z0d = jnp.where(cacc, acc0, zero16)
                zmd = jnp.where(cacc, accm * 0.125, zero16)
                plsc.store_scatter(z0_v, [isp, tsp, dsp, col], z0d)
                plsc.store_scatter(zm_v, [isp, tsp, dsp, col], zmd)
            return carry

        lax.fori_loop(0, ngrp * 2 * _NT, body, 0)

        pltpu.sync_copy(z0_v, z0_hbm.at[:, :, :, pl.ds(base, chunk)])
        pltpu.sync_copy(zm_v, zm_hbm.at[:, :, :, pl.ds(base, chunk)])

    return sc_main


def _make_post(B):
    def post_body(z0_ref, zm_ref, nodes_ref, embT_ref, wsemT_ref, bsem_ref,
                  qsem_ref, woutT_ref, bout_ref, v_ref, wt_ref, out_ref):
        def elu(x):
            return jnp.where(x > 0.0, x, jnp.exp(jnp.minimum(x, 0.0)) - 1.0)

        wsemT = wsemT_ref[...]
        bsem = bsem_ref[...]
        qsem = qsem_ref[...]
        v0 = v_ref[0, 0]
        vrest = jnp.sum(v_ref[...]) - v0
        wt = wt_ref[0, 0]
        iota_v = lax.broadcasted_iota(jnp.int32, (_VOCAB, B), 0)

        rs = []
        for i in range(2):
            out0_l, outm_l, wm_l = [], [], []
            for t in range(_NT):
                o0 = elu(z0_ref[i, t])
                om = elu(zm_ref[i, t])
                y0 = jnp.tanh(jnp.dot(wsemT, o0,
                                      preferred_element_type=jnp.float32) + bsem)
                ym = jnp.tanh(jnp.dot(wsemT, om,
                                      preferred_element_type=jnp.float32) + bsem)
                w0 = jnp.dot(qsem, y0, preferred_element_type=jnp.float32)
                wm = jnp.dot(qsem, ym, preferred_element_type=jnp.float32)
                wm_l.append((w0 + 7.0 * wm) * 0.125)
                out0_l.append(o0)
                outm_l.append(om)
            wmean = jnp.concatenate(wm_l, axis=0)
            mm = jnp.max(wmean, axis=0, keepdims=True)
            be = jnp.exp(wmean - mm)
            beta = be / jnp.sum(be, axis=0, keepdims=True)
            hp = beta[0:1] * (v0 * out0_l[0] + vrest * outm_l[0])
            for t in range(1, _NT):
                hp = hp + beta[t:t + 1] * (v0 * out0_l[t] + vrest * outm_l[t])
            hp = hp * wt
            oh = (iota_v == jnp.broadcast_to(nodes_ref[i:i + 1, :],
                                             (_VOCAB, B))).astype(jnp.float32)
            ne = jnp.dot(embT_ref[...], oh,
                         preferred_element_type=jnp.float32)
            ctx = jnp.concatenate([ne, hp], axis=0)
            r = jnp.dot(woutT_ref[...], ctx,
                        preferred_element_type=jnp.float32) + bout_ref[...]
            rs.append(r)

        num = jnp.sum(rs[0] * rs[1], axis=0, keepdims=True)
        n0 = jnp.sqrt(jnp.sum(rs[0] * rs[0], axis=0, keepdims=True))
        n1 = jnp.sqrt(jnp.sum(rs[1] * rs[1], axis=0, keepdims=True))
        out_ref[...] = num / jnp.maximum(n0 * n1, 1e-8)

    return pl.pallas_call(
        post_body,
        out_shape=jax.ShapeDtypeStruct((1, B), jnp.float32),
    )


@jax.jit
def _run(nodes, features, emb_table, W_gat, a_gat, W_sem, b_sem, q_sem,
         W_out, b_out, v, weight):
    B = nodes.shape[0]
    we, aux = pl.pallas_call(
        _tables_body,
        out_shape=[jax.ShapeDtypeStruct((_VOCAB, _D), jnp.float32),
                   jax.ShapeDtypeStruct((8, _VOCAB), jnp.float32)],
    )(emb_table, W_gat, a_gat.reshape(2, _D))

    nodes_t = nodes.T
    fp = jnp.transpose(features[:, :, :, 0, :], (1, 2, 3, 0))
    ids = jnp.concatenate(
        [jnp.broadcast_to(nodes_t[:, None, None, :], (2, _NT, 1, B)), fp],
        axis=2).reshape(2 * _NT * _NR, B)
    cids = jnp.transpose(features[:, :, :, 1:4, 0],
                         (1, 2, 3, 0)).reshape(2 * _NT * 3, B)

    z0, zm = _make_sc_main(B)(ids, cids, we, aux)

    sim = _make_post(B)(
        z0, zm, nodes_t, emb_table.T, W_sem.T, b_sem.reshape(_D, 1),
        q_sem.reshape(1, _D), W_out.T, b_out.reshape(300, 1),
        v.reshape(1, _NR), weight.reshape(1, 1))
    return sim.reshape(B)


def kernel(nodes, features, prop_nodes, prop_features, max_prop_len, emb_table,
           W_gat, a_gat, W_sem, b_sem, q_sem, W_out, b_out, v, weight):
    return _run(nodes, features, emb_table, W_gat, a_gat, W_sem, b_sem,
                q_sem, W_out, b_out, v, weight)
